# baseline (speedup 1.0000x reference)
import numpy as np
import jax
import jax.numpy as jnp
from jax import lax
from jax.experimental import pallas as pl
from jax.experimental.pallas import tpu as pltpu
from jax.experimental.pallas import tpu_sc as plsc

NUM_TABLES = 16
EMBED_DIM = 32
B, S = 4, 2048
P_TOTAL = B * S
NW = 32
PPW = P_TOTAL // NW
CHUNK_P = 8
N_CHUNKS = PPW // CHUNK_P

_SIZES = np.array([100000 + i for i in range(NUM_TABLES)], dtype=np.int64)
_OFFS = np.concatenate([[0], np.cumsum(_SIZES)[:-1]]).astype(np.int64)
_C32 = np.array([(1 << 32) % int(s) for s in _SIZES], dtype=np.int64)
TOTAL_ROWS = int(_SIZES.sum())
N_EMBD = 1024
SUP_W = 4 * EMBED_DIM


def _u32(x):
    return plsc.bitcast(x, jnp.uint32)


A_BLK = 64
A_FULL = 195
A_TAIL = 16
A_PER_W = A_FULL * A_BLK + A_TAIL
OUT_ROWS = 400032


def _shuffle(src_ref, dst_ref, nrows):
    for i in range(nrows * 8):
        o = 16 * i
        dst_ref[o // 128, pl.ds((o % 128), 16)] = (
            src_ref[o // 32, pl.ds((o % 32), 16)])


def _unpad_body(table_hbm, out_hbm, a0, a1, b0, b1, rsem, wsem):
    cid = lax.axis_index("c")
    sid = lax.axis_index("s")
    wid = sid * 2 + cid
    s_base = wid * A_PER_W
    abufs = [a0, a1]
    bbufs = [b0, b1]

    def _read(g, buf):
        s0 = s_base + g * A_BLK
        return pltpu.make_async_copy(
            table_hbm.at[pl.ds(pl.multiple_of(s0 * 4, 16), 4 * A_BLK)],
            abufs[buf], rsem[buf])

    def _write(g, buf):
        s0 = s_base + g * A_BLK
        return pltpu.make_async_copy(
            bbufs[buf], out_hbm.at[pl.ds(pl.multiple_of(s0, 16), A_BLK)],
            wsem[buf])

    _read(np.int32(0), 0).start()
    _read(np.int32(1), 1).start()
    n2 = (A_FULL - 1) // 2

    def dblock(_, g2):
        g = g2 * 2
        _read(g, 0).wait()

        @pl.when(g2 > 0)
        def _():
            _write(g - 2, 0).wait()

        _shuffle(a0, b0, A_BLK)
        _write(g, 0).start()
        _read(g + 2, 0).start()
        _read(g + 1, 1).wait()

        @pl.when(g2 > 0)
        def _():
            _write(g - 1, 1).wait()

        _shuffle(a1, b1, A_BLK)
        _write(g + 1, 1).start()

        @pl.when(g2 < n2 - 1)
        def _():
            _read(g + 3, 1).start()

        return g2 + 1

    lax.fori_loop(0, n2, dblock, np.int32(0))
    glast = np.int32(A_FULL - 1)
    _read(glast, 0).wait()
    _write(glast - 2, 0).wait()
    _shuffle(a0, b0, A_BLK)
    _write(glast, 0).start()
    _write(glast - 1, 1).wait()
    _write(glast, 0).wait()

    s0 = s_base + A_FULL * A_BLK
    pltpu.sync_copy(table_hbm.at[pl.ds(pl.multiple_of(s0 * 4, 16),
                                       4 * A_TAIL)],
                    a0.at[pl.ds(0, 4 * A_TAIL)])
    _shuffle(a0, b0, A_TAIL)
    pltpu.sync_copy(b0.at[pl.ds(0, A_TAIL)], out_hbm.at[pl.ds(s0, A_TAIL)])

    @pl.when(wid == NW - 1)
    def _():
        for s1, n, nsrc in ((399872, 64, 256), (399936, 64, 256),
                            (400000, 32, 120)):
            pltpu.sync_copy(table_hbm.at[pl.ds(4 * s1, nsrc)],
                            a1.at[pl.ds(0, nsrc)])
            _shuffle(a1, b1, n)
            pltpu.sync_copy(b1.at[pl.ds(0, n)],
                            out_hbm.at[pl.ds(s1, n)])


def _sc_unpad(table):
    mesh = plsc.VectorSubcoreMesh(core_axis_name="c", subcore_axis_name="s")
    fn = pl.kernel(
        _unpad_body,
        out_type=jax.ShapeDtypeStruct((OUT_ROWS, SUP_W), jnp.float32),
        mesh=mesh,
        scratch_types=[
            pltpu.VMEM((4 * A_BLK, EMBED_DIM), jnp.float32),
            pltpu.VMEM((4 * A_BLK, EMBED_DIM), jnp.float32),
            pltpu.VMEM((A_BLK, SUP_W), jnp.float32),
            pltpu.VMEM((A_BLK, SUP_W), jnp.float32),
            [pltpu.SemaphoreType.DMA, pltpu.SemaphoreType.DMA],
            [pltpu.SemaphoreType.DMA, pltpu.SemaphoreType.DMA],
        ],
    )
    return fn(table)


def _sc_body(table_hbm, tok_hbm, consts_hbm, s_hbm, m_hbm,
             tok_v, consts_v, idx_v, m_v, row_a, row_b, gsem):
    cid = lax.axis_index("c")
    sid = lax.axis_index("s")
    wid = sid * 2 + cid
    base = wid * PPW

    tok_off = pl.multiple_of(wid * (3 * PPW), 3 * PPW)
    pltpu.sync_copy(tok_hbm.at[pl.ds(tok_off, 3 * PPW)], tok_v)
    pltpu.sync_copy(consts_hbm, consts_v)

    def _cv(k):
        return _u32(consts_v[pl.ds(k * 16, 16)])

    ml0, mh0, ml1, mh1, ml2, mh2 = (_cv(k) for k in range(6))
    bias = _cv(6)
    sizes = _cv(7)
    c32 = _cv(8)
    offs = _cv(9)
    lane = lax.iota(jnp.int32, 16)
    is_tri = lane >= 8
    zero = jnp.zeros((16,), jnp.uint32)
    one = jnp.full((16,), 1, jnp.uint32)
    s16 = jnp.full((16,), 16, jnp.uint32)
    two = jnp.full((16,), 2, jnp.uint32)
    three = jnp.full((16,), 3, jnp.uint32)

    def _prod(ml, mh, tb):
        a = ml * tb
        bb = mh * tb
        lo = a + (bb << s16)
        carry = jnp.where(lo < a, one, zero)
        hi = (bb >> s16) + carry
        return lo, hi

    dnums = lax.GatherDimensionNumbers(
        offset_dims=(), collapsed_slice_dims=(0,), start_index_map=(0,))

    def _bcast(vec, j):
        jidx = jnp.full((16, 1), j, dtype=jnp.int32)
        return lax.gather(vec, jidx, dnums, (1,),
                          mode=lax.GatherScatterMode.PROMISE_IN_BOUNDS)

    def group_step(_, g):
        goff = g * 16
        t0g = tok_v[pl.ds(goff, 16)]
        t1g = tok_v[pl.ds(goff + PPW, 16)]
        t2g = tok_v[pl.ds(goff + 2 * PPW, 16)]
        for j in range(16):
            t0 = _u32(_bcast(t0g, j))
            t1 = _u32(_bcast(t1g, j))
            t2 = jnp.where(is_tri, _u32(_bcast(t2g, j)), zero)
            lo0, hi0 = _prod(ml0, mh0, t0)
            lo1, hi1 = _prod(ml1, mh1, t1)
            lo2, hi2 = _prod(ml2, mh2, t2)
            h_lo = lo0 ^ lo1 ^ lo2 ^ bias
            h_hi = hi0 ^ hi1 ^ hi2
            r = lax.rem(h_lo, sizes)
            acc = h_hi * c32 + r
            r2 = lax.rem(acc, sizes)
            full = r2 + offs
            sidx = plsc.bitcast(full >> two, jnp.int32)
            mcode = plsc.bitcast(full & three, jnp.int32)
            row = g * 2 + (j >> 3)
            col = (j & 7) * NUM_TABLES
            idx_v[row, pl.ds(col, NUM_TABLES)] = sidx
            m_v[goff + j, :] = mcode
        return g + 1

    lax.fori_loop(0, PPW // 16, group_step, np.int32(0))

    m_off = pl.multiple_of(base, PPW)
    pltpu.sync_copy(m_v, m_hbm.at[pl.ds(m_off, PPW)])

    i32 = jnp.int32
    rows = [row_a, row_b]
    descs = [None, None]
    descs[0] = pltpu.make_async_copy(
        table_hbm.at[idx_v.at[i32(0)]], rows[0], gsem[0])
    descs[0].start()
    for c in range(N_CHUNKS):
        buf = c % 2
        if c + 1 < N_CHUNKS:
            nbuf = (c + 1) % 2
            descs[nbuf] = pltpu.make_async_copy(
                table_hbm.at[idx_v.at[i32(c + 1)]], rows[nbuf], gsem[nbuf])
            descs[nbuf].start()
        descs[buf].wait()
        out_off = pl.multiple_of(
            base * NUM_TABLES + c * CHUNK_P * NUM_TABLES,
            CHUNK_P * NUM_TABLES)
        pltpu.sync_copy(
            rows[buf],
            s_hbm.at[pl.ds(out_off, CHUNK_P * NUM_TABLES)])


def _sc_gather(table4, tok_arr, consts):
    mesh = plsc.VectorSubcoreMesh(core_axis_name="c", subcore_axis_name="s")
    fn = pl.kernel(
        _sc_body,
        out_type=(
            jax.ShapeDtypeStruct((P_TOTAL * NUM_TABLES, SUP_W), jnp.float32),
            jax.ShapeDtypeStruct((P_TOTAL, NUM_TABLES), jnp.int32),
        ),
        mesh=mesh,
        scratch_types=[
            pltpu.VMEM((3 * PPW,), jnp.int32),
            pltpu.VMEM((160,), jnp.int32),
            pltpu.VMEM((N_CHUNKS, CHUNK_P * NUM_TABLES), jnp.int32),
            pltpu.VMEM((PPW, NUM_TABLES), jnp.int32),
            pltpu.VMEM((CHUNK_P * NUM_TABLES, SUP_W), jnp.float32),
            pltpu.VMEM((CHUNK_P * NUM_TABLES, SUP_W), jnp.float32),
            [pltpu.SemaphoreType.DMA, pltpu.SemaphoreType.DMA],
        ],
    )
    return fn(table4, tok_arr, consts)


M_BLK = 512


def _mm_body(x_ref, m_ref, w2_ref, o_ref):
    x = x_ref[...]
    m = m_ref[...]
    cols = lax.broadcasted_iota(jnp.int32, (M_BLK, NUM_TABLES * SUP_W), 1)
    kc = jnp.bitwise_and(jnp.right_shift(cols, 5), 3)
    m_exp = jnp.concatenate(
        [jnp.broadcast_to(m[:, t:t + 1], (M_BLK, SUP_W))
         for t in range(NUM_TABLES)], axis=1)
    xm = jnp.where(m_exp == kc, x, 0.0)
    o_ref[...] = lax.dot_general(
        xm, w2_ref[...], (((1,), (1,)), ((), ())),
        preferred_element_type=jnp.float32)


def _matmul(x, m, w2):
    grid = (P_TOTAL // M_BLK,)
    return pl.pallas_call(
        _mm_body,
        grid=grid,
        in_specs=[
            pl.BlockSpec((M_BLK, NUM_TABLES * SUP_W),
                         lambda i: (i, jnp.int32(0))),
            pl.BlockSpec((M_BLK, NUM_TABLES), lambda i: (i, jnp.int32(0))),
            pl.BlockSpec((N_EMBD, NUM_TABLES * SUP_W),
                         lambda i: (jnp.int32(0), jnp.int32(0))),
        ],
        out_specs=pl.BlockSpec((M_BLK, N_EMBD), lambda i: (i, jnp.int32(0))),
        out_shape=jax.ShapeDtypeStruct((P_TOTAL, N_EMBD), jnp.float32),
    )(x, m, w2)


def kernel(token_ids, table, W_out, hash_mults, hash_bias):
    tok32 = token_ids.astype(jnp.int32)
    sh0 = tok32
    sh1 = jnp.pad(tok32[:, :-1], ((0, 0), (1, 0)))
    sh2 = jnp.pad(tok32[:, :-2], ((0, 0), (2, 0)))
    stk = jnp.stack([sh0, sh1, sh2], axis=0)
    tok_arr = stk.reshape(3, NW, PPW).transpose(1, 0, 2).reshape(-1)

    mt = hash_mults.T
    ml = (mt & 0xFFFF).astype(jnp.int32)
    mh = (mt >> 16).astype(jnp.int32)
    consts = jnp.stack([
        ml[0], mh[0], ml[1], mh[1], ml[2], mh[2],
        hash_bias.astype(jnp.int32),
        jnp.asarray(_SIZES, jnp.int32),
        jnp.asarray(_C32, jnp.int32),
        jnp.asarray(_OFFS, jnp.int32),
    ]).reshape(-1)

    w2 = jnp.broadcast_to(
        W_out.reshape(N_EMBD, NUM_TABLES, 1, EMBED_DIM),
        (N_EMBD, NUM_TABLES, 4, EMBED_DIM),
    ).reshape(N_EMBD, NUM_TABLES * SUP_W)

    table4 = _sc_unpad(table)
    sup, mcodes = _sc_gather(table4, tok_arr, consts)
    x = sup.reshape(P_TOTAL, NUM_TABLES * SUP_W)
    out = _matmul(x, mcodes, w2)
    return out.reshape(B, S, N_EMBD)


# speedup vs baseline: 1.1731x; 1.1731x over previous
import numpy as np
import jax
import jax.numpy as jnp
from jax import lax
from jax.experimental import pallas as pl
from jax.experimental.pallas import tpu as pltpu
from jax.experimental.pallas import tpu_sc as plsc

NUM_TABLES = 16
EMBED_DIM = 32
B, S = 4, 2048
P_TOTAL = B * S
NW = 32
PPW = P_TOTAL // NW
CHUNK_P = 8
N_CHUNKS = PPW // CHUNK_P

_SIZES = np.array([100000 + i for i in range(NUM_TABLES)], dtype=np.int64)
_OFFS = np.concatenate([[0], np.cumsum(_SIZES)[:-1]]).astype(np.int64)
_C32 = np.array([(1 << 32) % int(s) for s in _SIZES], dtype=np.int64)
TOTAL_ROWS = int(_SIZES.sum())
N_EMBD = 1024


def _u32(x):
    return plsc.bitcast(x, jnp.uint32)


def _sc_body(table_hbm, tok_hbm, consts_hbm, emb_hbm,
             tok_v, consts_v, idx_v, row_v, gsem):
    cid = lax.axis_index("c")
    sid = lax.axis_index("s")
    wid = sid * 2 + cid
    base = wid * PPW

    pltpu.sync_copy(tok_hbm.at[pl.ds(wid * (3 * PPW), 3 * PPW)], tok_v)
    pltpu.sync_copy(consts_hbm, consts_v)

    ml0 = _u32(consts_v[0, :])
    mh0 = _u32(consts_v[1, :])
    ml1 = _u32(consts_v[2, :])
    mh1 = _u32(consts_v[3, :])
    ml2 = _u32(consts_v[4, :])
    mh2 = _u32(consts_v[5, :])
    bias = _u32(consts_v[6, :])
    sizes = _u32(consts_v[7, :])
    c32 = _u32(consts_v[8, :])
    offs = _u32(consts_v[9, :])
    lane = lax.iota(jnp.int32, 16)
    is_tri = lane >= 8
    zero = jnp.zeros((16,), jnp.uint32)
    one = jnp.full((16,), 1, jnp.uint32)
    s16 = jnp.full((16,), 16, jnp.uint32)

    def _prod(ml, mh, tb):
        a = ml * tb
        bb = mh * tb
        lo = a + (bb << s16)
        carry = jnp.where(lo < a, one, zero)
        hi = (bb >> s16) + carry
        return lo, hi

    dnums = lax.GatherDimensionNumbers(
        offset_dims=(), collapsed_slice_dims=(0,), start_index_map=(0,))

    def _bcast(vec, j):
        jidx = jnp.full((16, 1), j, dtype=jnp.int32)
        return lax.gather(vec, jidx, dnums, (1,),
                          mode=lax.GatherScatterMode.PROMISE_IN_BOUNDS)

    def group_step(_, g):
        goff = g * 16
        t0g = tok_v[pl.ds(goff, 16)]
        t1g = tok_v[pl.ds(goff + PPW, 16)]
        t2g = tok_v[pl.ds(goff + 2 * PPW, 16)]
        for j in range(16):
            t0 = _u32(_bcast(t0g, j))
            t1 = _u32(_bcast(t1g, j))
            t2 = jnp.where(is_tri, _u32(_bcast(t2g, j)), zero)
            lo0, hi0 = _prod(ml0, mh0, t0)
            lo1, hi1 = _prod(ml1, mh1, t1)
            lo2, hi2 = _prod(ml2, mh2, t2)
            h_lo = lo0 ^ lo1 ^ lo2 ^ bias
            h_hi = hi0 ^ hi1 ^ hi2
            r = lax.rem(h_lo, sizes)
            acc = h_hi * c32 + r
            r2 = lax.rem(acc, sizes)
            idx = plsc.bitcast(r2 + offs, jnp.int32)
            row = g * 2 + (j >> 3)
            col = (j & 7) * NUM_TABLES
            idx_v[row, pl.ds(col, NUM_TABLES)] = idx
        return g + 1

    lax.fori_loop(0, PPW // 16, group_step, np.int32(0))

    i32 = jnp.int32
    descs = [None, None]
    descs[0] = pltpu.make_async_copy(
        table_hbm.at[idx_v.at[i32(0)]], row_v.at[i32(0)], gsem[0])
    descs[0].start()
    for c in range(N_CHUNKS):
        buf = c % 2
        if c + 1 < N_CHUNKS:
            nbuf = (c + 1) % 2
            descs[nbuf] = pltpu.make_async_copy(
                table_hbm.at[idx_v.at[i32(c + 1)]], row_v.at[i32(nbuf)],
                gsem[nbuf])
            descs[nbuf].start()
        descs[buf].wait()
        pltpu.sync_copy(
            row_v.at[i32(buf)],
            emb_hbm.at[pl.ds(base * NUM_TABLES + c * CHUNK_P * NUM_TABLES,
                             CHUNK_P * NUM_TABLES)])


def _sc_gather(table, tok_arr, consts):
    mesh = plsc.VectorSubcoreMesh(core_axis_name="c", subcore_axis_name="s")
    fn = pl.kernel(
        _sc_body,
        out_type=jax.ShapeDtypeStruct((P_TOTAL * NUM_TABLES, EMBED_DIM),
                                      jnp.float32),
        mesh=mesh,
        scratch_types=[
            pltpu.VMEM((3 * PPW,), jnp.int32),
            pltpu.VMEM((10, 16), jnp.int32),
            pltpu.VMEM((N_CHUNKS, CHUNK_P * NUM_TABLES), jnp.int32),
            pltpu.VMEM((2, CHUNK_P * NUM_TABLES, EMBED_DIM), jnp.float32),
            [pltpu.SemaphoreType.DMA, pltpu.SemaphoreType.DMA],
        ],
        compiler_params=pltpu.CompilerParams(use_tc_tiling_on_sc=False),
    )
    return fn(table, tok_arr, consts)


def _mm_body(x_ref, w_ref, o_ref):
    o_ref[...] = lax.dot_general(
        x_ref[...], w_ref[...], (((1,), (1,)), ((), ())),
        preferred_element_type=jnp.float32)


def _matmul(x, w):
    m_blk = 1024
    grid = (x.shape[0] // m_blk,)
    return pl.pallas_call(
        _mm_body,
        grid=grid,
        in_specs=[
            pl.BlockSpec((m_blk, NUM_TABLES * EMBED_DIM),
                         lambda i: (i, jnp.int32(0))),
            pl.BlockSpec((N_EMBD, NUM_TABLES * EMBED_DIM),
                         lambda i: (jnp.int32(0), jnp.int32(0))),
        ],
        out_specs=pl.BlockSpec((m_blk, N_EMBD), lambda i: (i, jnp.int32(0))),
        out_shape=jax.ShapeDtypeStruct((x.shape[0], N_EMBD), jnp.float32),
    )(x, w)


def kernel(token_ids, table, W_out, hash_mults, hash_bias):
    tok32 = token_ids.astype(jnp.int32)
    sh0 = tok32
    sh1 = jnp.pad(tok32[:, :-1], ((0, 0), (1, 0)))
    sh2 = jnp.pad(tok32[:, :-2], ((0, 0), (2, 0)))
    stk = jnp.stack([sh0, sh1, sh2], axis=0)
    tok_arr = stk.reshape(3, NW, PPW).transpose(1, 0, 2).reshape(-1)

    mt = hash_mults.T
    ml = (mt & 0xFFFF).astype(jnp.int32)
    mh = (mt >> 16).astype(jnp.int32)
    consts = jnp.stack([
        ml[0], mh[0], ml[1], mh[1], ml[2], mh[2],
        hash_bias.astype(jnp.int32),
        jnp.asarray(_SIZES, jnp.int32),
        jnp.asarray(_C32, jnp.int32),
        jnp.asarray(_OFFS, jnp.int32),
    ])

    emb = _sc_gather(table, tok_arr, consts)
    x = emb.reshape(P_TOTAL, NUM_TABLES * EMBED_DIM)
    out = _matmul(x, W_out)
    return out.reshape(B, S, N_EMBD)
